# baseline (speedup 1.0000x reference)
import jax
import jax.numpy as jnp
from jax.experimental import pallas as pl
from jax.experimental.pallas import tpu as pltpu

_CHUNK = 512


def _body(t_hbm, tbl_hbm, out_hbm, t_v, tbl_v, buf0, buf1, sem_t, sem_tbl, sem_o0, sem_o1):
    nch = out_hbm.shape[0] // _CHUNK
    bufs = (buf0, buf1)
    osems = (sem_o0, sem_o1)
    ctbl = pltpu.make_async_copy(tbl_hbm, tbl_v, sem_tbl)
    ctbl.start()
    ct = pltpu.make_async_copy(t_hbm, t_v, sem_t)
    ct.start()
    ct.wait()
    v = tbl_v.shape[0]
    col = jax.lax.broadcasted_iota(jnp.int32, (_CHUNK, v), 1)
    writes = [None] * nch
    for c in range(nch):
        tb = t_v[pl.ds(c * _CHUNK, _CHUNK)].reshape(_CHUNK, 1)
        oh = (tb == col).astype(jnp.float32)
        if c == 0:
            ctbl.wait()
        if c >= 2:
            writes[c - 2].wait()
        bufs[c % 2][:, :] = jnp.dot(oh, tbl_v[:], preferred_element_type=jnp.float32)
        writes[c] = pltpu.make_async_copy(
            bufs[c % 2], out_hbm.at[pl.ds(c * _CHUNK, _CHUNK)], osems[c % 2]
        )
        writes[c].start()
    writes[nch - 2].wait()
    writes[nch - 1].wait()


def kernel(t, embed):
    B = t.shape[0]
    V, D = embed.shape
    t2 = t.astype(jnp.int32)
    return pl.pallas_call(
        _body,
        in_specs=[
            pl.BlockSpec(memory_space=pl.ANY),
            pl.BlockSpec(memory_space=pl.ANY),
        ],
        out_specs=pl.BlockSpec(memory_space=pl.ANY),
        out_shape=jax.ShapeDtypeStruct((B, D), jnp.float32),
        scratch_shapes=[
            pltpu.VMEM((B,), jnp.int32),
            pltpu.VMEM((V, D), jnp.float32),
            pltpu.VMEM((_CHUNK, D), jnp.float32),
            pltpu.VMEM((_CHUNK, D), jnp.float32),
            pltpu.SemaphoreType.DMA,
            pltpu.SemaphoreType.DMA,
            pltpu.SemaphoreType.DMA,
            pltpu.SemaphoreType.DMA,
        ],
    )(t2, embed)


# speedup vs baseline: 1.5216x; 1.5216x over previous
import jax
import jax.numpy as jnp
from jax.experimental import pallas as pl

_BB = 2048


def _lookup_block(t_ref, tbl_ref, out_ref):
    tb = t_ref[0, 0, :].reshape(_BB, 1)
    v = tbl_ref.shape[0]
    col = jax.lax.broadcasted_iota(jnp.int32, (_BB, v), 1)
    oh = (tb == col).astype(jnp.float32)
    out_ref[:, :] = jnp.dot(oh, tbl_ref[:], preferred_element_type=jnp.float32)


def kernel(t, embed):
    B = t.shape[0]
    V, D = embed.shape
    nb = B // _BB
    t3 = t.astype(jnp.int32).reshape(nb, 1, _BB)
    return pl.pallas_call(
        _lookup_block,
        grid=(nb,),
        in_specs=[
            pl.BlockSpec((1, 1, _BB), lambda i: (i, 0, 0)),
            pl.BlockSpec((V, D), lambda i: (0, 0)),
        ],
        out_specs=pl.BlockSpec((_BB, D), lambda i: (i, 0)),
        out_shape=jax.ShapeDtypeStruct((B, D), jnp.float32),
    )(t3, embed)
